# baseline (speedup 1.0000x reference)
import jax
import jax.numpy as jnp
from jax.experimental import pallas as pl
from jax.experimental.pallas import tpu as pltpu

_CHUNK_ROWS = 512
_NBUF = 16


def _router_body(x_hbm, w_ref, b_ref, out_ref, buf, sems):
    n = out_ref.shape[0]
    nchunks = n // _CHUNK_ROWS

    def start(chunk, slot):
        pltpu.make_async_copy(
            x_hbm.at[pl.ds(chunk * _CHUNK_ROWS, _CHUNK_ROWS), :],
            buf.at[slot],
            sems.at[slot],
        ).start()

    def wait(slot):
        pltpu.make_async_copy(
            x_hbm.at[pl.ds(0, _CHUNK_ROWS), :],
            buf.at[slot],
            sems.at[slot],
        ).wait()

    for s in range(min(_NBUF, nchunks)):
        start(s, s)

    def body(i, carry):
        slot = jax.lax.rem(i, _NBUF)
        wait(slot)
        x = buf[slot]
        logits = jax.lax.dot_general(
            x, w_ref[...], (((1,), (0,)), ((), ())),
            preferred_element_type=jnp.float32,
        ) + b_ref[...][None, :]
        m = jnp.max(logits, axis=-1, keepdims=True)
        e = jnp.exp(logits - m)
        out_ref[pl.ds(i * _CHUNK_ROWS, _CHUNK_ROWS), :] = (
            e / jnp.sum(e, axis=-1, keepdims=True)
        )
        nxt = i + _NBUF

        @pl.when(nxt < nchunks)
        def _():
            start(nxt, slot)

        return carry

    jax.lax.fori_loop(0, nchunks, body, 0)


def kernel(x_t, mem, W, b):
    n, d = x_t.shape
    n_exp = W.shape[1]
    probs = pl.pallas_call(
        _router_body,
        in_specs=[
            pl.BlockSpec(memory_space=pl.ANY),
            pl.BlockSpec(memory_space=pltpu.VMEM),
            pl.BlockSpec(memory_space=pltpu.VMEM),
        ],
        out_specs=pl.BlockSpec(memory_space=pltpu.VMEM),
        out_shape=jax.ShapeDtypeStruct((n, n_exp), jnp.float32),
        scratch_shapes=[
            pltpu.VMEM((_NBUF, _CHUNK_ROWS, d), jnp.float32),
            pltpu.SemaphoreType.DMA((_NBUF,)),
        ],
    )(x_t, W, b)
    return (probs, mem)
